# baseline (speedup 1.0000x reference)
import functools

import jax
import jax.numpy as jnp
from jax import lax
from jax.experimental import pallas as pl
from jax.experimental.pallas import tpu as pltpu
from jax.experimental.pallas import tpu_sc as plsc

N_NODES = 10000
N_EDGES = 10000
N_INC = 320000
D = 128

NC = 2
NS = 16
FC = 64
CP = 80
R_PAD = 10240
RPT = R_PAD // NS
K = 125
CHT = N_INC // NS // K
WCH = 16
NWIN = CHT // WCH
DIVB = 64
NDIV = RPT // DIVB

_mesh = plsc.VectorSubcoreMesh(
    core_axis_name="c", subcore_axis_name="s", num_cores=NC, num_subcores=NS)


@functools.partial(
    pl.kernel,
    out_type=jax.ShapeDtypeStruct((NC, R_PAD, CP), jnp.float32),
    mesh=_mesh,
    scratch_types=[
        pltpu.VMEM((WCH, K), jnp.int32),
        pltpu.VMEM((WCH, K), jnp.int32),
        pltpu.VMEM((K, CP), jnp.float32),
        pltpu.VMEM((DIVB, CP), jnp.float32),
        pltpu.VMEM_SHARED((R_PAD, CP), jnp.float32),
        pltpu.VMEM_SHARED((R_PAD, CP), jnp.float32),
    ],
    compiler_params=pltpu.CompilerParams(use_tc_tiling_on_sc=False),
)
def _sc_hyperconv(xs_hbm, nidx_hbm, eidx_hbm, zeros_hbm, out_hbm,
                  gidx_v, sidx_v, rows_v, div_v, buf_a, buf_b):
    cid = lax.axis_index("c")
    sid = lax.axis_index("s")
    tile_rows = pl.ds(sid * RPT, RPT)

    pltpu.sync_copy(xs_hbm.at[cid, tile_rows], buf_a.at[tile_rows])
    pltpu.sync_copy(zeros_hbm, buf_b.at[tile_rows])
    plsc.subcore_barrier()

    def _phase(src_sh, acc_sh, g_hbm, s_hbm):
        @pl.loop(0, NWIN)
        def _win(w):
            base = sid * CHT + w * WCH
            pltpu.sync_copy(g_hbm.at[pl.ds(base, WCH)], gidx_v)
            pltpu.sync_copy(s_hbm.at[pl.ds(base, WCH)], sidx_v)

            @pl.loop(0, WCH)
            def _chunk(j):
                pltpu.sync_copy(src_sh.at[gidx_v.at[j]], rows_v)
                pltpu.sync_copy(rows_v, acc_sh.at[sidx_v.at[j]], add=True)

    _phase(buf_a, buf_b, nidx_hbm, eidx_hbm)
    plsc.subcore_barrier()

    ones_block = jnp.full((16,), 1.0, jnp.float32)

    @pl.loop(0, NDIV)
    def _div(i):
        off = sid * RPT + i * DIVB
        pltpu.sync_copy(buf_b.at[pl.ds(off, DIVB)], div_v)

        @pl.loop(0, DIVB)
        def _row(r):
            cnt = div_v[r, pl.ds(FC, 16)]
            inv = 1.0 / jnp.maximum(cnt, 1.0)
            for k in range(FC // 16):
                div_v[r, pl.ds(k * 16, 16)] = div_v[r, pl.ds(k * 16, 16)] * inv
            div_v[r, pl.ds(FC, 16)] = ones_block

        pltpu.sync_copy(div_v, buf_b.at[pl.ds(off, DIVB)])

    pltpu.sync_copy(zeros_hbm, buf_a.at[tile_rows])
    plsc.subcore_barrier()

    _phase(buf_b, buf_a, eidx_hbm, nidx_hbm)
    plsc.subcore_barrier()

    pltpu.sync_copy(buf_a.at[tile_rows], out_hbm.at[cid, tile_rows])


_R = 1000


def _finish_body(agg_ref, w_ref, b_ref, out_ref):
    feats = jnp.concatenate([agg_ref[0, :, :FC], agg_ref[1, :, :FC]], axis=1)
    y = lax.dot_general(feats, w_ref[...], (((1,), (1,)), ((), ())),
                        preferred_element_type=jnp.float32)
    out_ref[...] = y + agg_ref[0, :, FC:FC + 1] * b_ref[...]


def kernel(x, hyperedge_index, W, b):
    ones = jnp.ones((NC, N_NODES, CP - FC), jnp.float32)
    halves = jnp.stack([x[:, :FC], x[:, FC:]], axis=0)
    xs = jnp.concatenate([halves, ones], axis=2)
    xs = jnp.concatenate(
        [xs, jnp.zeros((NC, R_PAD - N_NODES, CP), jnp.float32)], axis=1)

    nidx = hyperedge_index[0].reshape(NS * CHT, K)
    eidx = hyperedge_index[1].reshape(NS * CHT, K)
    zeros = jnp.zeros((RPT, CP), jnp.float32)

    agg = _sc_hyperconv(xs, nidx, eidx, zeros)

    out = pl.pallas_call(
        _finish_body,
        grid=(N_NODES // _R,),
        in_specs=[
            pl.BlockSpec((NC, _R, CP), lambda i: (0, i, 0)),
            pl.BlockSpec((D, D), lambda i: (0, 0)),
            pl.BlockSpec((1, D), lambda i: (0, 0)),
        ],
        out_specs=pl.BlockSpec((_R, D), lambda i: (i, 0)),
        out_shape=jax.ShapeDtypeStruct((N_NODES, D), jnp.float32),
    )(agg, W, b.reshape(1, D))
    return out


# speedup vs baseline: 1.2148x; 1.2148x over previous
import functools

import jax
import jax.numpy as jnp
from jax import lax
from jax.experimental import pallas as pl
from jax.experimental.pallas import tpu as pltpu
from jax.experimental.pallas import tpu_sc as plsc

N_NODES = 10000
N_EDGES = 10000
N_INC = 320000
D = 128
DP = 144

NC = 2
NS = 16
NW = NC * NS
K = 125
WCH = 8
NWIN = N_INC // NW // K // WCH
E_PAD = 10240
RPT = E_PAD // NS
CROWS = E_PAD // NW
CB = 40
NCB = CROWS // CB

_mesh = plsc.VectorSubcoreMesh(
    core_axis_name="c", subcore_axis_name="s", num_cores=NC, num_subcores=NS)


@functools.partial(
    pl.kernel,
    out_type=(
        jax.ShapeDtypeStruct((NC, E_PAD, DP), jnp.float32),
        jax.ShapeDtypeStruct((E_PAD, DP), jnp.float32),
        jax.ShapeDtypeStruct((NC, E_PAD, DP), jnp.float32),
    ),
    mesh=_mesh,
    scratch_types=[
        pltpu.VMEM((WCH, K), jnp.int32),
        pltpu.VMEM((WCH, K), jnp.int32),
        pltpu.VMEM((K, DP), jnp.float32),
        pltpu.VMEM((CB, DP), jnp.float32),
        pltpu.VMEM((CB, DP), jnp.float32),
        pltpu.VMEM_SHARED((E_PAD, DP), jnp.float32),
        pltpu.SemaphoreType.REGULAR,
    ],
    compiler_params=pltpu.CompilerParams(use_tc_tiling_on_sc=False),
)
def _sc_hyperconv(xp_hbm, nidx_hbm, eidx_hbm, zeros_hbm,
                  pa_hbm, ef_hbm, pb_hbm,
                  gidx_v, sidx_v, rows_v, cb0_v, cb1_v, acc_sh, xsem):
    cid = lax.axis_index("c")
    sid = lax.axis_index("s")
    wid = cid * NS + sid
    tile_rows = pl.ds(sid * RPT, RPT)

    def _xbarrier():
        plsc.subcore_barrier()

        @pl.when(sid == 0)
        def _():
            pl.semaphore_signal(xsem, 1, core_index=1 - cid)
            pl.semaphore_wait(xsem, 1)

        plsc.subcore_barrier()

    def _phase(src_hbm, g_hbm, s_hbm):
        @pl.loop(0, NWIN)
        def _win(w):
            base = wid * (NWIN * WCH) + w * WCH
            pltpu.sync_copy(g_hbm.at[pl.ds(base, WCH)], gidx_v)
            pltpu.sync_copy(s_hbm.at[pl.ds(base, WCH)], sidx_v)

            @pl.loop(0, WCH)
            def _chunk(j):
                pltpu.sync_copy(src_hbm.at[gidx_v.at[j]], rows_v)
                pltpu.sync_copy(rows_v, acc_sh.at[sidx_v.at[j]], add=True)

    pltpu.sync_copy(zeros_hbm, acc_sh.at[tile_rows])
    plsc.subcore_barrier()
    _phase(xp_hbm, nidx_hbm, eidx_hbm)

    plsc.subcore_barrier()
    pltpu.sync_copy(acc_sh.at[tile_rows], pa_hbm.at[cid, tile_rows])
    _xbarrier()

    ones16 = jnp.full((16,), 1.0, jnp.float32)

    @pl.loop(0, NCB)
    def _comb(i):
        off = wid * CROWS + i * CB
        pltpu.sync_copy(acc_sh.at[pl.ds(off, CB)], cb0_v)
        pltpu.sync_copy(pa_hbm.at[1 - cid, pl.ds(off, CB)], cb1_v)

        @pl.loop(0, CB)
        def _row(r):
            cnt = cb0_v[r, pl.ds(D, 16)] + cb1_v[r, pl.ds(D, 16)]
            inv = 1.0 / jnp.maximum(cnt, 1.0)
            for k in range(D // 16):
                s = cb0_v[r, pl.ds(k * 16, 16)] + cb1_v[r, pl.ds(k * 16, 16)]
                cb0_v[r, pl.ds(k * 16, 16)] = s * inv
            cb0_v[r, pl.ds(D, 16)] = ones16

        pltpu.sync_copy(cb0_v, ef_hbm.at[pl.ds(off, CB)])

    plsc.subcore_barrier()
    pltpu.sync_copy(zeros_hbm, acc_sh.at[tile_rows])
    _xbarrier()

    _phase(ef_hbm, eidx_hbm, nidx_hbm)
    plsc.subcore_barrier()
    pltpu.sync_copy(acc_sh.at[tile_rows], pb_hbm.at[cid, tile_rows])


_R = 1000


def _finish_body(agg_ref, w_ref, b_ref, out_ref):
    s = agg_ref[0] + agg_ref[1]
    y = lax.dot_general(s[:, :D], w_ref[...], (((1,), (1,)), ((), ())),
                        preferred_element_type=jnp.float32)
    out_ref[...] = y + s[:, D:D + 1] * b_ref[...]


def kernel(x, hyperedge_index, W, b):
    x_pad = jnp.concatenate([x, jnp.ones((N_NODES, DP - D), jnp.float32)],
                            axis=1)
    nidx = hyperedge_index[0].reshape(NW * NWIN * WCH, K)
    eidx = hyperedge_index[1].reshape(NW * NWIN * WCH, K)
    zeros = jnp.zeros((RPT, DP), jnp.float32)

    _, _, part_b = _sc_hyperconv(x_pad, nidx, eidx, zeros)

    out = pl.pallas_call(
        _finish_body,
        grid=(N_NODES // _R,),
        in_specs=[
            pl.BlockSpec((NC, _R, DP), lambda i: (0, i, 0)),
            pl.BlockSpec((D, D), lambda i: (0, 0)),
            pl.BlockSpec((1, D), lambda i: (0, 0)),
        ],
        out_specs=pl.BlockSpec((_R, D), lambda i: (i, 0)),
        out_shape=jax.ShapeDtypeStruct((N_NODES, D), jnp.float32),
    )(part_b, W, b.reshape(1, D))
    return out


# speedup vs baseline: 1.5311x; 1.2604x over previous
import functools

import jax
import jax.numpy as jnp
from jax import lax
from jax.experimental import pallas as pl
from jax.experimental.pallas import tpu as pltpu
from jax.experimental.pallas import tpu_sc as plsc

N_NODES = 10000
N_EDGES = 10000
N_INC = 320000
D = 128
DP = 144

NC = 2
NS = 16
NW = NC * NS
K = 125
WCH = 8
NWIN = N_INC // NW // K // WCH
E_PAD = 10240
RPT = E_PAD // NS
CROWS = E_PAD // NW
CB = 40
NCB = CROWS // CB

_mesh = plsc.VectorSubcoreMesh(
    core_axis_name="c", subcore_axis_name="s", num_cores=NC, num_subcores=NS)


@functools.partial(
    pl.kernel,
    out_type=(
        jax.ShapeDtypeStruct((NC, E_PAD, DP), jnp.float32),
        jax.ShapeDtypeStruct((E_PAD, DP), jnp.float32),
        jax.ShapeDtypeStruct((NC, E_PAD, DP), jnp.float32),
    ),
    mesh=_mesh,
    scratch_types=[
        pltpu.VMEM((WCH, K), jnp.int32),
        pltpu.VMEM((WCH, K), jnp.int32),
        pltpu.VMEM((K, DP), jnp.float32),
        pltpu.VMEM((K, DP), jnp.float32),
        pltpu.VMEM_SHARED((E_PAD, DP), jnp.float32),
        pltpu.SemaphoreType.REGULAR,
        pltpu.SemaphoreType.DMA,
        pltpu.SemaphoreType.DMA,
    ],
    compiler_params=pltpu.CompilerParams(use_tc_tiling_on_sc=False),
)
def _sc_hyperconv(xp_hbm, nidx_hbm, eidx_hbm, zeros_hbm,
                  pa_hbm, ef_hbm, pb_hbm,
                  gidx_v, sidx_v, rows0_v, rows1_v, acc_sh, xsem,
                  ssem0, ssem1):
    cid = lax.axis_index("c")
    sid = lax.axis_index("s")
    wid = cid * NS + sid
    tile_rows = pl.ds(sid * RPT, RPT)
    rows_b = (rows0_v, rows1_v)
    ssems = (ssem0, ssem1)
    GB = K * DP * 4

    def _xbarrier():
        plsc.subcore_barrier()

        @pl.when(sid == 0)
        def _():
            pl.semaphore_signal(xsem, 1, core_index=1 - cid)
            pl.semaphore_wait(xsem, 1)

        plsc.subcore_barrier()

    def _drain(t):
        pltpu.make_async_copy(xp_hbm.at[pl.ds(0, K)], rows_b[t],
                              ssems[t]).wait()

    def _phase(src_hbm, g_hbm, s_hbm):
        @pl.loop(0, NWIN)
        def _win(w):
            base = wid * (NWIN * WCH) + w * WCH
            pltpu.sync_copy(g_hbm.at[pl.ds(base, WCH)], gidx_v)
            pltpu.sync_copy(s_hbm.at[pl.ds(base, WCH)], sidx_v)

            @pl.loop(0, WCH, step=2)
            def _chunk(j):
                for t in range(2):
                    @pl.when(w + j > 0)
                    def _():
                        _drain(t)

                    pltpu.sync_copy(src_hbm.at[gidx_v.at[j + t]], rows_b[t])
                    pltpu.async_copy(rows_b[t], acc_sh.at[sidx_v.at[j + t]],
                                     ssems[t], add=True)

        _drain(0)
        _drain(1)

    pltpu.sync_copy(zeros_hbm, acc_sh.at[tile_rows])
    plsc.subcore_barrier()
    _phase(xp_hbm, nidx_hbm, eidx_hbm)

    plsc.subcore_barrier()
    pltpu.sync_copy(acc_sh.at[tile_rows], pa_hbm.at[cid, tile_rows])
    _xbarrier()

    ones16 = jnp.full((16,), 1.0, jnp.float32)

    @pl.loop(0, NCB)
    def _comb(i):
        off = wid * CROWS + i * CB
        pltpu.sync_copy(acc_sh.at[pl.ds(off, CB)], rows0_v.at[pl.ds(0, CB)])
        pltpu.sync_copy(pa_hbm.at[1 - cid, pl.ds(off, CB)],
                        rows1_v.at[pl.ds(0, CB)])

        @pl.loop(0, CB)
        def _row(r):
            cnt = rows0_v[r, pl.ds(D, 16)] + rows1_v[r, pl.ds(D, 16)]
            inv = 1.0 / jnp.maximum(cnt, 1.0)
            for k in range(D // 16):
                s = rows0_v[r, pl.ds(k * 16, 16)] + rows1_v[r, pl.ds(k * 16, 16)]
                rows0_v[r, pl.ds(k * 16, 16)] = s * inv
            rows0_v[r, pl.ds(D, 16)] = ones16

        pltpu.sync_copy(rows0_v.at[pl.ds(0, CB)], ef_hbm.at[pl.ds(off, CB)])

    plsc.subcore_barrier()
    pltpu.sync_copy(zeros_hbm, acc_sh.at[tile_rows])
    _xbarrier()

    _phase(ef_hbm, eidx_hbm, nidx_hbm)
    plsc.subcore_barrier()
    pltpu.sync_copy(acc_sh.at[tile_rows], pb_hbm.at[cid, tile_rows])


_R = 1000


def _finish_body(agg_ref, w_ref, b_ref, out_ref):
    s = agg_ref[0] + agg_ref[1]
    y = lax.dot_general(s[:, :D], w_ref[...], (((1,), (1,)), ((), ())),
                        preferred_element_type=jnp.float32)
    out_ref[...] = y + s[:, D:D + 1] * b_ref[...]


def kernel(x, hyperedge_index, W, b):
    x_pad = jnp.concatenate([x, jnp.ones((N_NODES, DP - D), jnp.float32)],
                            axis=1)
    nidx = hyperedge_index[0].reshape(NW * NWIN * WCH, K)
    eidx = hyperedge_index[1].reshape(NW * NWIN * WCH, K)
    zeros = jnp.zeros((RPT, DP), jnp.float32)

    _, _, part_b = _sc_hyperconv(x_pad, nidx, eidx, zeros)

    out = pl.pallas_call(
        _finish_body,
        grid=(N_NODES // _R,),
        in_specs=[
            pl.BlockSpec((NC, _R, DP), lambda i: (0, i, 0)),
            pl.BlockSpec((D, D), lambda i: (0, 0)),
            pl.BlockSpec((1, D), lambda i: (0, 0)),
        ],
        out_specs=pl.BlockSpec((_R, D), lambda i: (i, 0)),
        out_shape=jax.ShapeDtypeStruct((N_NODES, D), jnp.float32),
    )(part_b, W, b.reshape(1, D))
    return out
